# Initial kernel scaffold; baseline (speedup 1.0000x reference)
import jax
import jax.numpy as jnp
from jax.experimental import pallas as pl


def kernel(x, action_k, state_k):
    raise NotImplementedError("write your pallas kernel here")


# speedup vs baseline: 3.6950x; 3.6950x over previous
import functools

import jax
import jax.numpy as jnp
from jax import lax
from jax.experimental import pallas as pl
from jax.experimental.pallas import tpu as pltpu
from jax.experimental.pallas import tpu_sc as plsc

BSZ = 256
N = 10000
NCHUNK = N // 16
NWORKERS = 32
ROWS_PER_W = BSZ // NWORKERS
NBINS = 1024
BIN_SCALE = 512.0
CAND_CAP = N + 16
KPAD = 64
SENT_I = 1 << 20


@functools.partial(
    pl.kernel,
    out_type=[
        jax.ShapeDtypeStruct((BSZ * KPAD,), jnp.float32),
        jax.ShapeDtypeStruct((BSZ * KPAD,), jnp.int32),
    ],
    mesh=plsc.VectorSubcoreMesh(core_axis_name="c", subcore_axis_name="s"),
    compiler_params=pltpu.CompilerParams(needs_layout_passes=False),
    scratch_types=[
        pltpu.VMEM((2 * N,), jnp.float32),
        pltpu.VMEM((N,), jnp.float32),
        pltpu.VMEM((NBINS,), jnp.int32),
        pltpu.VMEM((CAND_CAP,), jnp.float32),
        pltpu.VMEM((CAND_CAP,), jnp.int32),
        pltpu.VMEM((ROWS_PER_W * KPAD,), jnp.float32),
        pltpu.VMEM((ROWS_PER_W * KPAD,), jnp.int32),
    ],
)
def _knn_sc(x_hbm, dist_out, idx_out, xrow, distb, hist, candd, candi,
            seld, seli):
    wid = lax.axis_index("s") * 2 + lax.axis_index("c")
    lanes = lax.iota(jnp.int32, 16)
    zeros16 = jnp.zeros((16,), jnp.int32)
    ones16 = jnp.ones((16,), jnp.int32)
    inf16 = jnp.full((16,), jnp.inf, jnp.float32)
    sent16 = jnp.full((16,), SENT_I, jnp.int32)
    INF = jnp.float32(jnp.inf)

    def do_row(r, _):
        b = wid * ROWS_PER_W + r
        pltpu.sync_copy(x_hbm.at[b], xrow)

        def hz(i, _):
            hist[pl.ds(i * 16, 16)] = zeros16
            return 0
        lax.fori_loop(0, NBINS // 16, hz, 0)

        v0 = xrow[pl.ds(0, 16)]
        qx = jnp.broadcast_to(jnp.min(jnp.where(lanes == 0, v0, inf16)),
                              (16,))
        qy = jnp.broadcast_to(jnp.min(jnp.where(lanes == 1, v0, inf16)),
                              (16,))

        def pass_a(ci, _):
            pidx = 2 * (ci * 16 + lanes)
            xs = plsc.load_gather(xrow, [pidx])
            ys = plsc.load_gather(xrow, [pidx + 1])
            dx = xs - qx
            dy = ys - qy
            d = dx * dx + dy * dy
            distb[pl.ds(ci * 16, 16)] = d
            bins = jnp.minimum((d * BIN_SCALE).astype(jnp.int32), NBINS - 1)
            plsc.addupdate_scatter(hist, [bins], ones16)
            return 0
        lax.fori_loop(0, NCHUNK, pass_a, 0)

        def scan_bin(v, carry):
            bbin, running = carry
            hv = hist[pl.ds(v * 16, 16)]
            c = jnp.cumsum(hv)
            tot = jnp.max(c)
            cum = running + c
            anyhit = (running + tot) >= 50
            first = jnp.sum((cum < 50).astype(jnp.int32))
            newb = jnp.where((bbin == SENT_I) & anyhit, v * 16 + first, bbin)
            return newb, running + tot
        bbin, _ = lax.fori_loop(0, NBINS // 16, scan_bin,
                                (jnp.int32(SENT_I), jnp.int32(0)))

        def pass_b(ci, pos):
            d = distb[pl.ds(ci * 16, 16)]
            bins = jnp.minimum((d * BIN_SCALE).astype(jnp.int32), NBINS - 1)
            m = bins <= bbin
            off = jnp.cumsum(m.astype(jnp.int32))
            tgt = pos + off - 1
            plsc.store_scatter(candd, [tgt], d, mask=m)
            plsc.store_scatter(candi, [tgt], ci * 16 + lanes, mask=m)
            return pos + jnp.max(off)
        cnum = lax.fori_loop(0, NCHUNK, pass_b, jnp.int32(0))

        plsc.store_scatter(candd, [cnum + lanes], inf16)
        plsc.store_scatter(candi, [cnum + lanes], sent16)
        nv = (cnum + 15) // 16

        def rank_loop(rk, _):
            def scanv(v, carry):
                bd, bi = carry
                dv = candd[pl.ds(v * 16, 16)]
                iv = candi[pl.ds(v * 16, 16)]
                better = (dv < bd) | ((dv == bd) & (iv < bi))
                return (jnp.where(better, dv, bd),
                        jnp.where(better, iv, bi))
            bd, bi = lax.fori_loop(0, nv, scanv, (inf16, sent16))
            m = jnp.min(bd)
            mi = jnp.min(jnp.where(bd == m, bi, sent16))

            lane0 = lanes == 0
            o = jnp.broadcast_to(r * KPAD + rk, (16,))
            plsc.store_scatter(seld, [o], jnp.broadcast_to(m, (16,)),
                               mask=lane0)
            plsc.store_scatter(seli, [o], jnp.broadcast_to(mi, (16,)),
                               mask=lane0)

            def inval(v, _):
                dv = candd[pl.ds(v * 16, 16)]
                iv = candi[pl.ds(v * 16, 16)]
                w = (dv == m) & (iv == mi)
                candd[pl.ds(v * 16, 16)] = jnp.where(w, INF, dv)
                return 0
            lax.fori_loop(0, nv, inval, 0)
            return 0
        lax.fori_loop(0, 50, rank_loop, 0)
        return 0

    lax.fori_loop(0, ROWS_PER_W, do_row, 0)
    base = wid * ROWS_PER_W * KPAD
    pltpu.sync_copy(seld, dist_out.at[pl.ds(base, ROWS_PER_W * KPAD)])
    pltpu.sync_copy(seli, idx_out.at[pl.ds(base, ROWS_PER_W * KPAD)])


def kernel(x, action_k, state_k):
    xf = x.reshape(BSZ, 2 * N)
    dist_flat, idx_flat = _knn_sc(xf)
    dist_pad = dist_flat.reshape(BSZ, KPAD)
    idx_pad = idx_flat.reshape(BSZ, KPAD)
    knn_dist = dist_pad[:, :50]
    knn_idx = idx_pad[:, :50]
    return knn_dist, knn_idx, knn_idx[:, :20]


# speedup vs baseline: 4.5159x; 1.2222x over previous
import functools

import jax
import jax.numpy as jnp
from jax import lax
from jax.experimental import pallas as pl
from jax.experimental.pallas import tpu as pltpu
from jax.experimental.pallas import tpu_sc as plsc

BSZ = 256
N = 10000
NCHUNK = N // 16
NCPAD = 640
NWORKERS = 32
ROWS_PER_W = BSZ // NWORKERS
NBINS = 1024
BIN_SCALE = 65536.0
CAND_CAP = N + 16
KPAD = 64
SENT_I = 1 << 20


@functools.partial(
    pl.kernel,
    out_type=[
        jax.ShapeDtypeStruct((BSZ * KPAD,), jnp.float32),
        jax.ShapeDtypeStruct((BSZ * KPAD,), jnp.int32),
    ],
    mesh=plsc.VectorSubcoreMesh(core_axis_name="c", subcore_axis_name="s"),
    compiler_params=pltpu.CompilerParams(needs_layout_passes=False),
    scratch_types=[
        pltpu.VMEM((2 * N,), jnp.float32),
        pltpu.VMEM((NCPAD,), jnp.float32),
        pltpu.VMEM((NCPAD,), jnp.int32),
        pltpu.VMEM((NBINS,), jnp.int32),
        pltpu.VMEM((CAND_CAP,), jnp.float32),
        pltpu.VMEM((CAND_CAP,), jnp.int32),
        pltpu.VMEM((ROWS_PER_W * KPAD,), jnp.float32),
        pltpu.VMEM((ROWS_PER_W * KPAD,), jnp.int32),
    ],
)
def _knn_sc(x_hbm, dist_out, idx_out, xrow, cmin, clist, hist, candd, candi,
            seld, seli):
    wid = lax.axis_index("s") * 2 + lax.axis_index("c")
    lanes = lax.iota(jnp.int32, 16)
    zeros16 = jnp.zeros((16,), jnp.int32)
    ones16 = jnp.ones((16,), jnp.int32)
    inf16 = jnp.full((16,), jnp.inf, jnp.float32)
    sent16 = jnp.full((16,), SENT_I, jnp.int32)
    INF = jnp.float32(jnp.inf)
    lane0 = lanes == 0

    def do_row(r, _):
        b = wid * ROWS_PER_W + r
        pltpu.sync_copy(x_hbm.at[b], xrow)

        def hz(i, _):
            hist[pl.ds(i * 16, 16)] = zeros16
            return 0
        lax.fori_loop(0, NBINS // 16, hz, 0, unroll=4)
        cmin[pl.ds(NCHUNK - 1, 16)] = inf16

        v0 = xrow[pl.ds(0, 16)]
        qx = jnp.broadcast_to(jnp.min(jnp.where(lanes == 0, v0, inf16)),
                              (16,))
        qy = jnp.broadcast_to(jnp.min(jnp.where(lanes == 1, v0, inf16)),
                              (16,))

        def pass_a(ci, _):
            pidx = 2 * (ci * 16 + lanes)
            xs = plsc.load_gather(xrow, [pidx])
            ys = plsc.load_gather(xrow, [pidx + 1])
            dx = xs - qx
            dy = ys - qy
            d = dx * dx + dy * dy
            bins = jnp.minimum((d * BIN_SCALE).astype(jnp.int32), NBINS - 1)
            plsc.addupdate_scatter(hist, [bins], ones16)
            plsc.store_scatter(cmin, [jnp.broadcast_to(ci, (16,))],
                               jnp.broadcast_to(jnp.min(d), (16,)),
                               mask=lane0)
            return 0
        lax.fori_loop(0, NCHUNK, pass_a, 0, unroll=5)

        def scan_bin(v, carry):
            bbin, running = carry
            hv = hist[pl.ds(v * 16, 16)]
            c = jnp.cumsum(hv)
            tot = jnp.max(c)
            cum = running + c
            anyhit = (running + tot) >= 50
            first = jnp.sum((cum < 50).astype(jnp.int32))
            newb = jnp.where((bbin == SENT_I) & anyhit, v * 16 + first, bbin)
            return newb, running + tot
        bbin, _ = lax.fori_loop(0, NBINS // 16, scan_bin,
                                (jnp.int32(SENT_I), jnp.int32(0)), unroll=2)

        def chunk_scan(v, pos):
            cm = cmin[pl.ds(v * 16, 16)]
            cb = jnp.minimum((cm * BIN_SCALE).astype(jnp.int32), NBINS - 1)
            m = (cb <= bbin) & (v * 16 + lanes < NCHUNK)
            off = jnp.cumsum(m.astype(jnp.int32))
            plsc.store_scatter(clist, [pos + off - 1], v * 16 + lanes,
                               mask=m)
            return pos + jnp.max(off)
        nvisit = lax.fori_loop(0, NCPAD // 16, chunk_scan, jnp.int32(0),
                               unroll=4)

        def pass_b(j, pos):
            civ = plsc.load_gather(clist, [jnp.broadcast_to(j, (16,))])
            pidx = 2 * (civ * 16 + lanes)
            xs = plsc.load_gather(xrow, [pidx])
            ys = plsc.load_gather(xrow, [pidx + 1])
            dx = xs - qx
            dy = ys - qy
            d = dx * dx + dy * dy
            bins = jnp.minimum((d * BIN_SCALE).astype(jnp.int32), NBINS - 1)
            m = bins <= bbin
            off = jnp.cumsum(m.astype(jnp.int32))
            tgt = pos + off - 1
            plsc.store_scatter(candd, [tgt], d, mask=m)
            plsc.store_scatter(candi, [tgt], civ * 16 + lanes, mask=m)
            return pos + jnp.max(off)
        cnum = lax.fori_loop(0, nvisit, pass_b, jnp.int32(0))

        plsc.store_scatter(candd, [cnum + lanes], inf16)
        plsc.store_scatter(candi, [cnum + lanes], sent16)
        nv = (cnum + 15) // 16

        def rank_loop(rk, _):
            def scanv(v, carry):
                bd, bi = carry
                dv = candd[pl.ds(v * 16, 16)]
                iv = candi[pl.ds(v * 16, 16)]
                better = (dv < bd) | ((dv == bd) & (iv < bi))
                return (jnp.where(better, dv, bd),
                        jnp.where(better, iv, bi))
            bd, bi = lax.fori_loop(0, nv, scanv, (inf16, sent16))
            m = jnp.min(bd)
            mi = jnp.min(jnp.where(bd == m, bi, sent16))

            o = jnp.broadcast_to(r * KPAD + rk, (16,))
            plsc.store_scatter(seld, [o], jnp.broadcast_to(m, (16,)),
                               mask=lane0)
            plsc.store_scatter(seli, [o], jnp.broadcast_to(mi, (16,)),
                               mask=lane0)

            def inval(v, _):
                dv = candd[pl.ds(v * 16, 16)]
                iv = candi[pl.ds(v * 16, 16)]
                w = (dv == m) & (iv == mi)
                candd[pl.ds(v * 16, 16)] = jnp.where(w, INF, dv)
                return 0
            lax.fori_loop(0, nv, inval, 0)
            return 0
        lax.fori_loop(0, 50, rank_loop, 0)
        return 0

    lax.fori_loop(0, ROWS_PER_W, do_row, 0)
    base = wid * ROWS_PER_W * KPAD
    pltpu.sync_copy(seld, dist_out.at[pl.ds(base, ROWS_PER_W * KPAD)])
    pltpu.sync_copy(seli, idx_out.at[pl.ds(base, ROWS_PER_W * KPAD)])


def kernel(x, action_k, state_k):
    xf = x.reshape(BSZ, 2 * N)
    dist_flat, idx_flat = _knn_sc(xf)
    dist_pad = dist_flat.reshape(BSZ, KPAD)
    idx_pad = idx_flat.reshape(BSZ, KPAD)
    knn_dist = dist_pad[:, :50]
    knn_idx = idx_pad[:, :50]
    return knn_dist, knn_idx, knn_idx[:, :20]


# speedup vs baseline: 5.5819x; 1.2361x over previous
import functools

import jax
import jax.numpy as jnp
from jax import lax
from jax.experimental import pallas as pl
from jax.experimental.pallas import tpu as pltpu
from jax.experimental.pallas import tpu_sc as plsc

BSZ = 256
N = 10000
NWORKERS = 32
ROWS_PER_W = BSZ // NWORKERS
NSUPER = 125
NBINS = 1024
BIN_SCALE = 65536.0
CAND_CAP = N + 64
KPAD = 64
SENT_I = 1 << 20


def _lex_lt(ad, ai, bd, bi):
    return (ad < bd) | ((ad == bd) & (ai < bi))


def _cx(ad, ai, bd, bi):
    lt = _lex_lt(ad, ai, bd, bi)
    return (jnp.where(lt, ad, bd), jnp.where(lt, ai, bi),
            jnp.where(lt, bd, ad), jnp.where(lt, bi, ai))


def _rev2(d, i):
    return lax.rev(d, (0,)), lax.rev(i, (0,))


def _clean16(lanes, d, i):
    for j in (8, 4, 2, 1):
        pj = jnp.bitwise_xor(lanes, j)
        pd = jnp.take_along_axis(d, pj, axis=0)
        pi = jnp.take_along_axis(i, pj, axis=0)
        plt = _lex_lt(pd, pi, d, i)
        mind = jnp.where(plt, pd, d)
        mini = jnp.where(plt, pi, i)
        maxd = jnp.where(plt, d, pd)
        maxi = jnp.where(plt, i, pi)
        low = (lanes & j) == 0
        d = jnp.where(low, mind, maxd)
        i = jnp.where(low, mini, maxi)
    return d, i


def _merge16(lanes, ad, ai, bd, bi):
    rbd, rbi = _rev2(bd, bi)
    ld, li, hd, hi = _cx(ad, ai, rbd, rbi)
    ld, li = _clean16(lanes, ld, li)
    hd, hi = _clean16(lanes, hd, hi)
    return ld, li, hd, hi


def _merge32(lanes, a, b):
    a1d, a1i, a2d, a2i = a
    b1d, b1i, b2d, b2i = b
    r1d, r1i = _rev2(b2d, b2i)
    r2d, r2i = _rev2(b1d, b1i)
    l1d, l1i, h1d, h1i = _cx(a1d, a1i, r1d, r1i)
    l2d, l2i, h2d, h2i = _cx(a2d, a2i, r2d, r2i)
    lo1d, lo1i, lo2d, lo2i = _cx(l1d, l1i, l2d, l2i)
    hi1d, hi1i, hi2d, hi2i = _cx(h1d, h1i, h2d, h2i)
    out = []
    for dd, ii in ((lo1d, lo1i), (lo2d, lo2i), (hi1d, hi1i), (hi2d, hi2i)):
        out.extend(_clean16(lanes, dd, ii))
    return tuple(out)


def _merge64_low(lanes, s, g):
    sd = s[0::2]
    si = s[1::2]
    gd = g[0::2]
    gi = g[1::2]
    lows = []
    for k in range(4):
        rgd, rgi = _rev2(gd[3 - k], gi[3 - k])
        lt = _lex_lt(sd[k], si[k], rgd, rgi)
        lows.append((jnp.where(lt, sd[k], rgd), jnp.where(lt, si[k], rgi)))
    l0, l1, l2, l3 = lows
    a0d, a0i, a2d, a2i = _cx(l0[0], l0[1], l2[0], l2[1])
    a1d, a1i, a3d, a3i = _cx(l1[0], l1[1], l3[0], l3[1])
    b0d, b0i, b1d, b1i = _cx(a0d, a0i, a1d, a1i)
    b2d, b2i, b3d, b3i = _cx(a2d, a2i, a3d, a3i)
    out = []
    for dd, ii in ((b0d, b0i), (b1d, b1i), (b2d, b2i), (b3d, b3i)):
        out.extend(_clean16(lanes, dd, ii))
    return tuple(out)


@functools.partial(
    pl.kernel,
    out_type=[
        jax.ShapeDtypeStruct((BSZ * KPAD,), jnp.float32),
        jax.ShapeDtypeStruct((BSZ * KPAD,), jnp.int32),
    ],
    mesh=plsc.VectorSubcoreMesh(core_axis_name="c", subcore_axis_name="s"),
    compiler_params=pltpu.CompilerParams(needs_layout_passes=False),
    scratch_types=[
        pltpu.VMEM((2 * N,), jnp.float32),
        pltpu.VMEM((128,), jnp.float32),
        pltpu.VMEM((128,), jnp.int32),
        pltpu.VMEM((NBINS,), jnp.int32),
        pltpu.VMEM((CAND_CAP,), jnp.float32),
        pltpu.VMEM((CAND_CAP,), jnp.int32),
        pltpu.VMEM((ROWS_PER_W * KPAD,), jnp.float32),
        pltpu.VMEM((ROWS_PER_W * KPAD,), jnp.int32),
    ],
)
def _knn_sc(x_hbm, dist_out, idx_out, xrow, qmin, clist, hist, candd, candi,
            seld, seli):
    wid = lax.axis_index("s") * 2 + lax.axis_index("c")
    lanes = lax.iota(jnp.int32, 16)
    zeros16 = jnp.zeros((16,), jnp.int32)
    ones16 = jnp.ones((16,), jnp.int32)
    inf16 = jnp.full((16,), jnp.inf, jnp.float32)
    sent16 = jnp.full((16,), SENT_I, jnp.int32)
    lane0 = lanes == 0

    def do_row(r, _):
        b = wid * ROWS_PER_W + r
        pltpu.sync_copy(x_hbm.at[b], xrow)

        def hz(i, _):
            hist[pl.ds(i * 16, 16)] = zeros16
            return 0
        lax.fori_loop(0, NBINS // 16, hz, 0, unroll=4)
        qmin[pl.ds(112, 16)] = inf16

        v0 = xrow[pl.ds(0, 16)]
        qx = jnp.broadcast_to(jnp.min(jnp.where(lanes == 0, v0, inf16)),
                              (16,))
        qy = jnp.broadcast_to(jnp.min(jnp.where(lanes == 1, v0, inf16)),
                              (16,))

        def pass_a(sc, _):
            base2 = sc * 160
            mv = inf16
            for k in range(5):
                pidx = base2 + 2 * (k * 16 + lanes)
                xs = plsc.load_gather(xrow, [pidx])
                ys = plsc.load_gather(xrow, [pidx + 1])
                dx = xs - qx
                dy = ys - qy
                d = dx * dx + dy * dy
                bins = jnp.minimum((d * BIN_SCALE).astype(jnp.int32),
                                   NBINS - 1)
                plsc.addupdate_scatter(hist, [bins], ones16,
                                       mask=bins < NBINS - 1)
                mv = jnp.minimum(mv, d)
            for j in (8, 4, 2, 1):
                mv = jnp.minimum(
                    mv, jnp.take_along_axis(mv, jnp.bitwise_xor(lanes, j),
                                            axis=0))
            plsc.store_scatter(qmin, [jnp.broadcast_to(sc, (16,))], mv,
                               mask=lane0)
            return 0
        lax.fori_loop(0, NSUPER, pass_a, 0, unroll=2)

        def hsum(t, acc):
            return acc + plsc.load_gather(hist, [lanes * 64 + t])
        acc = lax.fori_loop(0, 64, hsum, zeros16, unroll=8)
        cumacc = jnp.cumsum(acc)
        g = jnp.sum((cumacc < 50).astype(jnp.int32))
        cnt_bef = jnp.sum(jnp.where(lanes < g, acc, 0))
        gq = jnp.minimum(g, 15)

        def fine(t, carry):
            bbin_c, running = carry
            hv = hist[pl.ds(gq * 64 + t * 16, 16)]
            c = jnp.cumsum(hv)
            tot = jnp.max(c)
            cum = running + c
            anyhit = (running + tot) >= 50
            first = jnp.sum((cum < 50).astype(jnp.int32))
            newb = jnp.where((bbin_c == SENT_I) & anyhit,
                             gq * 64 + t * 16 + first, bbin_c)
            return newb, running + tot
        bbin, _ = lax.fori_loop(0, 4, fine,
                                (jnp.int32(SENT_I), cnt_bef), unroll=4)
        bbin = jnp.where(g >= 16, NBINS - 1, bbin)

        def sscan(v, pos):
            qm = qmin[pl.ds(v * 16, 16)]
            qb = jnp.minimum((qm * BIN_SCALE).astype(jnp.int32), NBINS - 1)
            m = (qb <= bbin) & (v * 16 + lanes < NSUPER)
            off = jnp.cumsum(m.astype(jnp.int32))
            plsc.store_scatter(clist, [pos + off - 1], v * 16 + lanes,
                               mask=m)
            return pos + jnp.max(off)
        nvisit = lax.fori_loop(0, 8, sscan, jnp.int32(0), unroll=4)

        def pass_b(j, pos):
            scv = plsc.load_gather(clist, [jnp.broadcast_to(j, (16,))])
            base2 = scv * 160
            for k in range(5):
                pidx = base2 + 2 * (k * 16 + lanes)
                xs = plsc.load_gather(xrow, [pidx])
                ys = plsc.load_gather(xrow, [pidx + 1])
                dx = xs - qx
                dy = ys - qy
                d = dx * dx + dy * dy
                bins = jnp.minimum((d * BIN_SCALE).astype(jnp.int32),
                                   NBINS - 1)
                m = bins <= bbin
                off = jnp.cumsum(m.astype(jnp.int32))
                tgt = pos + off - 1
                plsc.store_scatter(candd, [tgt], d, mask=m)
                plsc.store_scatter(candi, [tgt],
                                   scv * 80 + k * 16 + lanes, mask=m)
                pos = pos + jnp.max(off)
            return pos
        cnum = lax.fori_loop(0, nvisit, pass_b, jnp.int32(0))

        for t in range(4):
            plsc.store_scatter(candd, [cnum + t * 16 + lanes], inf16)
            plsc.store_scatter(candi, [cnum + t * 16 + lanes], sent16)

        def grp(gidx, s):
            base = gidx * 64
            runs = []
            for t in range(4):
                dv = candd[pl.ds(base + t * 16, 16)]
                iv = candi[pl.ds(base + t * 16, 16)]
                sd, si = lax.sort((dv, iv), dimension=0, num_keys=1)
                runs.append((sd, si))
            a = _merge16(lanes, *runs[0], *runs[1])
            c = _merge16(lanes, *runs[2], *runs[3])
            gg = _merge32(lanes, a, c)
            return _merge64_low(lanes, s, gg)
        s0 = (inf16, sent16) * 4
        ngrp = (cnum + 63) // 64
        s = lax.fori_loop(0, ngrp, grp, s0)

        for t in range(4):
            seld[pl.ds(r * KPAD + t * 16, 16)] = s[2 * t]
            seli[pl.ds(r * KPAD + t * 16, 16)] = s[2 * t + 1]
        return 0

    lax.fori_loop(0, ROWS_PER_W, do_row, 0)
    base = wid * ROWS_PER_W * KPAD
    pltpu.sync_copy(seld, dist_out.at[pl.ds(base, ROWS_PER_W * KPAD)])
    pltpu.sync_copy(seli, idx_out.at[pl.ds(base, ROWS_PER_W * KPAD)])


def kernel(x, action_k, state_k):
    xf = x.reshape(BSZ, 2 * N)
    dist_flat, idx_flat = _knn_sc(xf)
    dist_pad = dist_flat.reshape(BSZ, KPAD)
    idx_pad = idx_flat.reshape(BSZ, KPAD)
    knn_dist = dist_pad[:, :50]
    knn_idx = idx_pad[:, :50]
    return knn_dist, knn_idx, knn_idx[:, :20]
